# Initial kernel scaffold; baseline (speedup 1.0000x reference)
import jax
import jax.numpy as jnp
from jax.experimental import pallas as pl


def kernel(node_reps, mask, in_indices, in_edges, in_mask, out_indices, out_edges, out_mask, edge_table, W_nb, b_nb, W_node, b_node, attenW):
    raise NotImplementedError("write your pallas kernel here")


# speedup vs baseline: 4.9114x; 4.9114x over previous
import functools

import jax
import jax.numpy as jnp
from jax import lax
from jax.experimental import pallas as pl
from jax.experimental.pallas import tpu as pltpu
from jax.experimental.pallas import tpu_sc as plsc

N = 10000
K = 32
D = 128
D2 = 2 * D
V = 1000
L = 16

NW = 32
NPW = 320
NPAD = NW * NPW
SCN = 32
CH = 4
NCH = SCN // CH
NSC = NPW // SCN
UNR = 8

_mesh = plsc.VectorSubcoreMesh(core_axis_name="c", subcore_axis_name="s")
_NC = 2


@functools.partial(
    pl.kernel,
    mesh=_mesh,
    compiler_params=pltpu.CompilerParams(needs_layout_passes=False),
    out_type=jax.ShapeDtypeStruct((NPAD, D2), jnp.float32),
    scratch_types=[
        pltpu.VMEM((2, NCH, CH * K), jnp.int32),
        pltpu.VMEM((2, NCH, CH * K), jnp.int32),
        pltpu.VMEM((2, NCH, CH * K), jnp.float32),
        pltpu.VMEM((SCN, D2), jnp.float32),
        pltpu.VMEM((CH * K, D), jnp.float32),
        pltpu.VMEM((CH * K, D), jnp.float32),
        pltpu.VMEM((CH, D2), jnp.float32),
        pltpu.SemaphoreType.DMA,
        pltpu.SemaphoreType.DMA,
    ],
)
def _sc_gat(node_hbm, q_hbm, etab_hbm,
            iidx_hbm, iedg_hbm, imsk_hbm,
            oidx_hbm, oedg_hbm, omsk_hbm,
            s_hbm,
            idx_v, edg_v, msk_v, q_v, g_v, e_v, s_v, sem_g, sem_e):
    wid = lax.axis_index("s") * _NC + lax.axis_index("c")
    dirs = ((iidx_hbm, iedg_hbm, imsk_hbm), (oidx_hbm, oedg_hbm, omsk_hbm))

    def superchunk(scc, _):
        nbase = wid * NPW + scc * SCN
        crow = wid * (NPW // CH) + scc * NCH
        for dd in range(2):
            ih, eh, mh = dirs[dd]
            pltpu.sync_copy(ih.at[pl.ds(crow, NCH)], idx_v.at[dd])
            pltpu.sync_copy(eh.at[pl.ds(crow, NCH)], edg_v.at[dd])
            pltpu.sync_copy(mh.at[pl.ds(crow, NCH)], msk_v.at[dd])
        pltpu.sync_copy(q_hbm.at[pl.ds(nbase, SCN)], q_v)

        def chunk(c, _):
            for dd in range(2):
                cp_g = pltpu.async_copy(node_hbm.at[idx_v.at[dd, c]], g_v, sem_g)
                cp_e = pltpu.async_copy(etab_hbm.at[edg_v.at[dd, c]], e_v, sem_e)
                cp_g.wait()
                cp_e.wait()

                def node_body(j, _, dd=dd):
                    nl = c * CH + j
                    row0 = j * K
                    q1 = [q_v[nl, pl.ds(t * L, L)] for t in range(8)]
                    q2 = [q_v[nl, pl.ds(D + t * L, L)] for t in range(8)]

                    lane = lax.broadcasted_iota(jnp.int32, (L,), 0)
                    z = jnp.zeros((L,), jnp.float32)
                    lv = [z, z]
                    for k in range(K):
                        r = row0 + k
                        acc = g_v[r, pl.ds(0, L)] * q1[0]
                        for t in range(1, 8):
                            acc = acc + g_v[r, pl.ds(t * L, L)] * q1[t]
                        for t in range(8):
                            acc = acc + e_v[r, pl.ds(t * L, L)] * q2[t]
                        h, pos = divmod(k, L)
                        lv[h] = jnp.where(lane == pos, jnp.sum(acc), lv[h])

                    mx = jnp.max(jnp.maximum(lv[0], lv[1]))
                    w0 = jnp.exp(lv[0] - mx)
                    w1 = jnp.exp(lv[1] - mx)
                    tot = jnp.sum(w0) + jnp.sum(w1)
                    mk0 = msk_v[dd, c, pl.ds(j * K, L)]
                    mk1 = msk_v[dd, c, pl.ds(j * K + L, L)]
                    w0 = w0 / tot * mk0
                    w1 = w1 / tot * mk1

                    s1 = [z] * 8
                    s2 = [z] * 8
                    for k in range(K):
                        r = row0 + k
                        h, pos = divmod(k, L)
                        wk = (w0 if h == 0 else w1)[pos]
                        for t in range(8):
                            s1[t] = s1[t] + wk * g_v[r, pl.ds(t * L, L)]
                            s2[t] = s2[t] + wk * e_v[r, pl.ds(t * L, L)]
                    for t in range(8):
                        if dd == 0:
                            s_v[j, pl.ds(t * L, L)] = s1[t]
                            s_v[j, pl.ds(D + t * L, L)] = s2[t]
                        else:
                            s_v[j, pl.ds(t * L, L)] += s1[t]
                            s_v[j, pl.ds(D + t * L, L)] += s2[t]
                    return 0

                lax.fori_loop(0, CH, node_body, 0, unroll=False)
            pltpu.sync_copy(s_v, s_hbm.at[pl.ds(nbase + c * CH, CH)])
            return 0

        lax.fori_loop(0, NCH, chunk, 0, unroll=False)
        return 0

    lax.fori_loop(0, NSC, superchunk, 0, unroll=False)


def _q_tc(node_ref, aw_ref, wn_ref, q_ref):
    p = lax.dot_general(aw_ref[...], wn_ref[...], (((0,), (0,)), ((), ())),
                        preferred_element_type=jnp.float32)
    q_ref[...] = lax.dot_general(node_ref[...], p, (((1,), (0,)), ((), ())),
                                 preferred_element_type=jnp.float32)


def _o_tc(node_ref, s_ref, w_ref, b_ref, o_ref):
    acc = lax.dot_general(s_ref[...], w_ref[...], (((1,), (1,)), ((), ())),
                          preferred_element_type=jnp.float32)
    o_ref[...] = node_ref[...] + acc + 2.0 * b_ref[...]


def kernel(node_reps, mask, in_indices, in_edges, in_mask,
           out_indices, out_edges, out_mask,
           edge_table, W_nb, b_nb, W_node, b_node, attenW):
    node_sq = node_reps[0]
    node_pad = jnp.concatenate(
        [node_sq, jnp.zeros((NPAD - N, D), jnp.float32)], axis=0)

    def prep(a):
        return jnp.pad(a[0], ((0, NPAD - N), (0, 0))).reshape(NPAD // CH, CH * K)

    iidx, iedg, imsk = prep(in_indices), prep(in_edges), prep(in_mask)
    oidx, oedg, omsk = prep(out_indices), prep(out_edges), prep(out_mask)

    QB = 1024
    q_pad = pl.pallas_call(
        _q_tc,
        grid=(NPAD // QB,),
        in_specs=[pl.BlockSpec((QB, D), lambda i: (i, 0)),
                  pl.BlockSpec((D, D), lambda i: (0, 0)),
                  pl.BlockSpec((D, D2), lambda i: (0, 0))],
        out_specs=pl.BlockSpec((QB, D2), lambda i: (i, 0)),
        out_shape=jax.ShapeDtypeStruct((NPAD, D2), jnp.float32),
    )(node_pad, attenW, W_node)

    s = _sc_gat(node_sq, q_pad, edge_table,
                iidx, iedg, imsk, oidx, oedg, omsk)

    OB = 1000
    out = pl.pallas_call(
        _o_tc,
        grid=(N // OB,),
        in_specs=[pl.BlockSpec((OB, D), lambda i: (i, 0)),
                  pl.BlockSpec((OB, D2), lambda i: (i, 0)),
                  pl.BlockSpec((D, D2), lambda i: (0, 0)),
                  pl.BlockSpec((1, D), lambda i: (0, 0))],
        out_specs=pl.BlockSpec((OB, D), lambda i: (i, 0)),
        out_shape=jax.ShapeDtypeStruct((N, D), jnp.float32),
    )(node_sq, s[:N], W_nb, b_nb.reshape(1, D))

    return out[None]
